# baseline (speedup 1.0000x reference)
import functools

import jax
import jax.numpy as jnp
from jax import lax
from jax.experimental import pallas as pl
from jax.experimental.pallas import tpu as pltpu
from jax.experimental.pallas import tpu_sc as plsc

NUM_EMB = 8192
DIM = 64
M_TILE = 256
N_CHUNK = 4096

_NC, _NS = 2, 16
_NW = _NC * _NS
_IDX_CHUNK = 128


def _row_sumsq(zt):
    q = (zt * zt).reshape(M_TILE, 8, 8)
    a = q[:, 0, :]
    for t in range(1, 8):
        a = a + q[:, t, :]
    b = a[:, 0:4] + a[:, 4:8]
    c = b[:, 0:2] + b[:, 2:4]
    return c[:, 0:1] + c[:, 1:2]


def _argmin_body(zt_ref, cbt_ref, idx_ref, sum_ref, minv_ref, mini_ref):
    n = pl.program_id(1)
    zt = zt_ref[...]
    cbt = cbt_ref[...]
    zsq = _row_sumsq(zt)
    csq = jnp.sum(cbt * cbt, axis=0, keepdims=True)
    mm = lax.dot_general(zt.astype(jnp.bfloat16), cbt.astype(jnp.bfloat16),
                         (((1,), (0,)), ((), ())),
                         preferred_element_type=jnp.float32)
    dist = (zsq + csq) - 2.0 * mm
    d3 = dist.reshape(M_TILE, N_CHUNK // 128, 128)
    best_v = d3[:, 0, :]
    best_c = jnp.zeros((M_TILE, 128), jnp.int32)
    for c in range(1, N_CHUNK // 128):
        dv = d3[:, c, :]
        lt = dv < best_v
        best_v = jnp.where(lt, dv, best_v)
        best_c = jnp.where(lt, c, best_c)
    gidx = best_c * 128 + lax.broadcasted_iota(jnp.int32, (M_TILE, 128), 1)
    local_min = jnp.min(best_v, axis=1)
    masked = jnp.where(best_v == local_min[:, None], gidx, N_CHUNK)
    local_arg = jnp.min(masked, axis=1)

    @pl.when(n == 0)
    def _():
        minv_ref[...] = local_min
        mini_ref[...] = local_arg

    @pl.when(n == 1)
    def _():
        m1 = minv_ref[...]
        carry = m1.astype(jnp.bfloat16).astype(jnp.float32)
        take2 = local_min < carry
        idx_ref[...] = jnp.where(take2, local_arg + N_CHUNK, mini_ref[...])
        s = jnp.sum(jnp.where(take2, local_min, m1))

        @pl.when(pl.program_id(0) == 0)
        def _():
            sum_ref[0, 0] = s

        @pl.when(pl.program_id(0) > 0)
        def _():
            sum_ref[0, 0] += s


def _argmin_call(zf, cbt):
    m_tiles = zf.shape[0] // M_TILE
    return pl.pallas_call(
        _argmin_body,
        grid=(m_tiles, NUM_EMB // N_CHUNK),
        in_specs=[
            pl.BlockSpec((M_TILE, DIM), lambda m, n: (m, 0)),
            pl.BlockSpec((DIM, N_CHUNK), lambda m, n: (0, n)),
        ],
        out_specs=[
            pl.BlockSpec((M_TILE,), lambda m, n: (m,)),
            pl.BlockSpec(memory_space=pltpu.SMEM),
        ],
        out_shape=[
            jax.ShapeDtypeStruct((zf.shape[0],), jnp.int32),
            jax.ShapeDtypeStruct((1, 1), jnp.float32),
        ],
        scratch_shapes=[
            pltpu.VMEM((M_TILE,), jnp.float32),
            pltpu.VMEM((M_TILE,), jnp.int32),
        ],
    )(zf, cbt)


def _gather_rows(codebook, idx):
    b = idx.shape[0]
    b_per_w = b // _NW
    mesh = plsc.VectorSubcoreMesh(core_axis_name="c", subcore_axis_name="s")

    @functools.partial(
        pl.kernel,
        out_type=jax.ShapeDtypeStruct((b, DIM), jnp.float32),
        mesh=mesh,
        scratch_types=[
            pltpu.VMEM((b_per_w,), jnp.int32),
            pltpu.VMEM((b_per_w, DIM), jnp.float32),
            pltpu.SemaphoreType.DMA,
        ],
        compiler_params=pltpu.CompilerParams(use_tc_tiling_on_sc=False),
    )
    def gk(cb_hbm, idx_hbm, out_hbm, idx_v, rows_v, sem):
        wid = lax.axis_index("s") * _NC + lax.axis_index("c")
        base = wid * b_per_w
        pltpu.sync_copy(idx_hbm.at[pl.ds(base, b_per_w)], idx_v)
        for c in range(b_per_w // _IDX_CHUNK):
            pltpu.async_copy(
                cb_hbm.at[idx_v.at[pl.ds(c * _IDX_CHUNK, _IDX_CHUNK)]],
                rows_v.at[pl.ds(c * _IDX_CHUNK, _IDX_CHUNK)],
                sem,
            ).wait()
        pltpu.sync_copy(rows_v, out_hbm.at[pl.ds(base, b_per_w)])

    return gk(codebook, idx)


def kernel(z, codebook):
    b, l, d = z.shape
    zf = z.reshape(-1, d)
    idx_flat, dist_sum = _argmin_call(zf, codebook.T)
    z_q = _gather_rows(codebook, idx_flat).reshape(b, l, d)
    vq_loss = 1.25 * (dist_sum[0, 0] / (b * l * d))
    return (z_q, idx_flat.reshape(b, l), vq_loss)


# speedup vs baseline: 2.9667x; 2.9667x over previous
import functools

import jax
import jax.numpy as jnp
from jax import lax
from jax.experimental import pallas as pl
from jax.experimental.pallas import tpu as pltpu
from jax.experimental.pallas import tpu_sc as plsc

NUM_EMB = 8192
DIM = 64
M_TILE = 256
N_CHUNK = 4096

_NC, _NS = 2, 16
_NW = _NC * _NS
_IDX_CHUNK = 128


def _row_sumsq(zt):
    q = (zt * zt).reshape(M_TILE, 8, 8)
    a = q[:, 0, :]
    for t in range(1, 8):
        a = a + q[:, t, :]
    b = a[:, 0:4] + a[:, 4:8]
    c = b[:, 0:2] + b[:, 2:4]
    return c[:, 0:1] + c[:, 1:2]


def _argmin_body(zt_ref, cbt_ref, csq_ref, idx_ref, sum_ref, minv_ref,
                 mini_ref, zsqb_ref):
    n = pl.program_id(1)
    zt = zt_ref[...]
    zsq = _row_sumsq(zt)
    zsqb_ref[...] = jnp.broadcast_to(zsq, (M_TILE, 128))
    zsq_bc = zsqb_ref[...]
    csq8 = csq_ref[...]
    mm = lax.dot_general(zt.astype(jnp.bfloat16), cbt_ref[...],
                         (((1,), (0,)), ((), ())),
                         preferred_element_type=jnp.float32)
    lane = lax.broadcasted_iota(jnp.int32, (8, 128), 1)
    mins, args = [], []
    for rb in range(M_TILE // 8):
        zv = zsq_bc[rb * 8:(rb + 1) * 8, :]
        bv = bc = None
        for c in range(N_CHUNK // 128):
            sl = slice(c * 128, (c + 1) * 128)
            dv = (zv + csq8[:, sl]) - 2.0 * mm[rb * 8:(rb + 1) * 8, sl]
            if c == 0:
                bv, bc = dv, jnp.zeros((8, 128), jnp.int32)
            else:
                lt = dv < bv
                bv = jnp.where(lt, dv, bv)
                bc = jnp.where(lt, c, bc)
        gidx = bc * 128 + lane
        lm = jnp.min(bv, axis=1)
        ma = jnp.where(bv == lm[:, None], gidx, N_CHUNK)
        mins.append(lm)
        args.append(jnp.min(ma, axis=1))
    local_min = jnp.concatenate(mins)
    local_arg = jnp.concatenate(args)

    @pl.when(n == 0)
    def _():
        minv_ref[...] = local_min
        mini_ref[...] = local_arg

    @pl.when(n == 1)
    def _():
        m1 = minv_ref[...]
        carry = m1.astype(jnp.bfloat16).astype(jnp.float32)
        take2 = local_min < carry
        idx_ref[...] = jnp.where(take2, local_arg + N_CHUNK, mini_ref[...])
        s = jnp.sum(jnp.where(take2, local_min, m1))

        @pl.when(pl.program_id(0) == 0)
        def _():
            sum_ref[0, 0] = s

        @pl.when(pl.program_id(0) > 0)
        def _():
            sum_ref[0, 0] += s


def _argmin_call(zf, cbt_bf16, csq):
    m_tiles = zf.shape[0] // M_TILE
    return pl.pallas_call(
        _argmin_body,
        grid=(m_tiles, NUM_EMB // N_CHUNK),
        in_specs=[
            pl.BlockSpec((M_TILE, DIM), lambda m, n: (m, 0)),
            pl.BlockSpec((DIM, N_CHUNK), lambda m, n: (0, n)),
            pl.BlockSpec((8, N_CHUNK), lambda m, n: (0, n)),
        ],
        out_specs=[
            pl.BlockSpec((M_TILE,), lambda m, n: (m,)),
            pl.BlockSpec(memory_space=pltpu.SMEM),
        ],
        out_shape=[
            jax.ShapeDtypeStruct((zf.shape[0],), jnp.int32),
            jax.ShapeDtypeStruct((1, 1), jnp.float32),
        ],
        scratch_shapes=[
            pltpu.VMEM((M_TILE,), jnp.float32),
            pltpu.VMEM((M_TILE,), jnp.int32),
            pltpu.VMEM((M_TILE, 128), jnp.float32),
        ],
    )(zf, cbt_bf16, csq)


def _gather_rows(codebook, idx):
    b = idx.shape[0]
    b_per_w = b // _NW
    mesh = plsc.VectorSubcoreMesh(core_axis_name="c", subcore_axis_name="s")

    @functools.partial(
        pl.kernel,
        out_type=jax.ShapeDtypeStruct((b, DIM), jnp.float32),
        mesh=mesh,
        scratch_types=[
            pltpu.VMEM((b_per_w,), jnp.int32),
            pltpu.VMEM((b_per_w, DIM), jnp.float32),
            pltpu.SemaphoreType.DMA,
        ],
        compiler_params=pltpu.CompilerParams(use_tc_tiling_on_sc=False),
    )
    def gk(cb_hbm, idx_hbm, out_hbm, idx_v, rows_v, sem):
        wid = lax.axis_index("s") * _NC + lax.axis_index("c")
        base = wid * b_per_w
        pltpu.sync_copy(idx_hbm.at[pl.ds(base, b_per_w)], idx_v)
        for c in range(b_per_w // _IDX_CHUNK):
            pltpu.async_copy(
                cb_hbm.at[idx_v.at[pl.ds(c * _IDX_CHUNK, _IDX_CHUNK)]],
                rows_v.at[pl.ds(c * _IDX_CHUNK, _IDX_CHUNK)],
                sem,
            ).wait()
        pltpu.sync_copy(rows_v, out_hbm.at[pl.ds(base, b_per_w)])

    return gk(codebook, idx)


def kernel(z, codebook):
    b, l, d = z.shape
    zf = z.reshape(-1, d)
    csq = jnp.broadcast_to(
        jnp.sum(codebook * codebook, axis=1)[None, :], (8, NUM_EMB))
    idx_flat, dist_sum = _argmin_call(
        zf, codebook.T.astype(jnp.bfloat16), csq)
    z_q = _gather_rows(codebook, idx_flat).reshape(b, l, d)
    vq_loss = 1.25 * (dist_sum[0, 0] / (b * l * d))
    return (z_q, idx_flat.reshape(b, l), vq_loss)


# speedup vs baseline: 5.0832x; 1.7134x over previous
import functools

import jax
import jax.numpy as jnp
from jax import lax
from jax.experimental import pallas as pl
from jax.experimental.pallas import tpu as pltpu
from jax.experimental.pallas import tpu_sc as plsc

NUM_EMB = 8192
DIM = 64
M_TILE = 512
N_CHUNK = 4096

_NC, _NS = 2, 16
_NW = _NC * _NS
_IDX_CHUNK = 128


def _row_sumsq(zt):
    q = (zt * zt).reshape(M_TILE, 8, 8)
    a = q[:, 0, :]
    for t in range(1, 8):
        a = a + q[:, t, :]
    b = a[:, 0:4] + a[:, 4:8]
    c = b[:, 0:2] + b[:, 2:4]
    return c[:, 0:1] + c[:, 1:2]


def _argmin_body(zt_ref, cbt_ref, csq_ref, idx_ref, sum_ref, minv_ref,
                 mini_ref, zsqb_ref):
    n = pl.program_id(1)
    @pl.when(n == 0)
    def _():
        zsq = _row_sumsq(zt_ref[...])
        zsqb_ref[...] = jnp.broadcast_to(zsq, (M_TILE, 128))

    zsq_bc = zsqb_ref[...]
    csq8 = csq_ref[...]
    mm = lax.dot_general(zt_ref[...].astype(jnp.bfloat16), cbt_ref[...],
                         (((1,), (0,)), ((), ())),
                         preferred_element_type=jnp.float32)
    lane = lax.broadcasted_iota(jnp.int32, (8, 128), 1)
    mins, args = [], []
    for rb in range(M_TILE // 8):
        zv = zsq_bc[rb * 8:(rb + 1) * 8, :]
        bv = bc = None
        for c in range(N_CHUNK // 128):
            sl = slice(c * 128, (c + 1) * 128)
            dv = (zv + csq8[:, sl]) - 2.0 * mm[rb * 8:(rb + 1) * 8, sl]
            if c == 0:
                bv, bc = dv, jnp.zeros((8, 128), jnp.int32)
            else:
                lt = dv < bv
                bv = jnp.where(lt, dv, bv)
                bc = jnp.where(lt, c, bc)
        gidx = bc * 128 + lane
        lm = jnp.min(bv, axis=1)
        ma = jnp.where(bv == lm[:, None], gidx, N_CHUNK)
        mins.append(lm)
        args.append(jnp.min(ma, axis=1))
    local_min = jnp.concatenate(mins)
    local_arg = jnp.concatenate(args)

    @pl.when(n == 0)
    def _():
        minv_ref[...] = local_min
        mini_ref[...] = local_arg

    @pl.when(n == 1)
    def _():
        m1 = minv_ref[...]
        carry = m1.astype(jnp.bfloat16).astype(jnp.float32)
        take2 = local_min < carry
        idx_ref[...] = jnp.where(take2, local_arg + N_CHUNK, mini_ref[...])
        s = jnp.sum(jnp.where(take2, local_min, m1))

        @pl.when(pl.program_id(0) == 0)
        def _():
            sum_ref[0, 0] = s

        @pl.when(pl.program_id(0) > 0)
        def _():
            sum_ref[0, 0] += s


def _argmin_call(zf, cbt_bf16, csq):
    m_tiles = zf.shape[0] // M_TILE
    return pl.pallas_call(
        _argmin_body,
        grid=(m_tiles, NUM_EMB // N_CHUNK),
        in_specs=[
            pl.BlockSpec((M_TILE, DIM), lambda m, n: (m, 0)),
            pl.BlockSpec((DIM, N_CHUNK), lambda m, n: (0, n)),
            pl.BlockSpec((8, N_CHUNK), lambda m, n: (0, n)),
        ],
        out_specs=[
            pl.BlockSpec((M_TILE,), lambda m, n: (m,)),
            pl.BlockSpec(memory_space=pltpu.SMEM),
        ],
        out_shape=[
            jax.ShapeDtypeStruct((zf.shape[0],), jnp.int32),
            jax.ShapeDtypeStruct((1, 1), jnp.float32),
        ],
        scratch_shapes=[
            pltpu.VMEM((M_TILE,), jnp.float32),
            pltpu.VMEM((M_TILE,), jnp.int32),
            pltpu.VMEM((M_TILE, 128), jnp.float32),
        ],
    )(zf, cbt_bf16, csq)


def _gather_rows(codebook, idx):
    b = idx.shape[0]
    b_per_w = b // _NW
    mesh = plsc.VectorSubcoreMesh(core_axis_name="c", subcore_axis_name="s")

    @functools.partial(
        pl.kernel,
        out_type=jax.ShapeDtypeStruct((b, DIM), jnp.float32),
        mesh=mesh,
        scratch_types=[
            pltpu.VMEM((b_per_w,), jnp.int32),
            pltpu.VMEM((b_per_w, DIM), jnp.float32),
            pltpu.SemaphoreType.DMA,
        ],
        compiler_params=pltpu.CompilerParams(use_tc_tiling_on_sc=False),
    )
    def gk(cb_hbm, idx_hbm, out_hbm, idx_v, rows_v, sem):
        wid = lax.axis_index("s") * _NC + lax.axis_index("c")
        base = wid * b_per_w
        pltpu.sync_copy(idx_hbm.at[pl.ds(base, b_per_w)], idx_v)
        for c in range(b_per_w // _IDX_CHUNK):
            pltpu.async_copy(
                cb_hbm.at[idx_v.at[pl.ds(c * _IDX_CHUNK, _IDX_CHUNK)]],
                rows_v.at[pl.ds(c * _IDX_CHUNK, _IDX_CHUNK)],
                sem,
            ).wait()
        pltpu.sync_copy(rows_v, out_hbm.at[pl.ds(base, b_per_w)])

    return gk(codebook, idx)


def kernel(z, codebook):
    b, l, d = z.shape
    zf = z.reshape(-1, d)
    csq = jnp.broadcast_to(
        jnp.sum(codebook * codebook, axis=1)[None, :], (8, NUM_EMB))
    idx_flat, dist_sum = _argmin_call(
        zf, codebook.T.astype(jnp.bfloat16), csq)
    z_q = _gather_rows(codebook, idx_flat).reshape(b, l, d)
    vq_loss = 1.25 * (dist_sum[0, 0] / (b * l * d))
    return (z_q, idx_flat.reshape(b, l), vq_loss)


# speedup vs baseline: 6.7476x; 1.3274x over previous
import functools

import jax
import jax.numpy as jnp
from jax import lax
from jax.experimental import pallas as pl
from jax.experimental.pallas import tpu as pltpu
from jax.experimental.pallas import tpu_sc as plsc

NUM_EMB = 8192
DIM = 64
M_TILE = 512
N_CHUNK = 4096

_NC, _NS = 2, 16
_NW = _NC * _NS
_IDX_CHUNK = 128


def _row_sumsq_t(ztt):
    q = ztt * ztt
    a = q[0:8, :]
    for t in range(1, 8):
        a = a + q[8 * t:8 * (t + 1), :]
    b = a[0:4, :] + a[4:8, :]
    c = b[0:2, :] + b[2:4, :]
    return c[0:1, :] + c[1:2, :]


def _argmin_body(zt_ref, cbt_ref, csq_ref, ztt_ref, idx_ref, sum_ref,
                 minv_ref, mini_ref, zsqb_ref):
    n = pl.program_id(1)
    @pl.when(n == 0)
    def _():
        zsq = _row_sumsq_t(ztt_ref[...]).reshape(M_TILE, 1)
        zsqb_ref[...] = jnp.broadcast_to(zsq, (M_TILE, 128))

    zsq_bc = zsqb_ref[...]
    csq8 = csq_ref[...]
    mm = lax.dot_general(zt_ref[...].astype(jnp.bfloat16), cbt_ref[...],
                         (((1,), (0,)), ((), ())),
                         preferred_element_type=jnp.float32)
    lane = lax.broadcasted_iota(jnp.int32, (8, 128), 1)
    mins, args = [], []
    for rb in range(M_TILE // 8):
        zv = zsq_bc[rb * 8:(rb + 1) * 8, :]
        bv = bc = None
        for c in range(N_CHUNK // 128):
            sl = slice(c * 128, (c + 1) * 128)
            dv = (zv + csq8[:, sl]) - 2.0 * mm[rb * 8:(rb + 1) * 8, sl]
            if c == 0:
                bv, bc = dv, jnp.zeros((8, 128), jnp.int32)
            else:
                lt = dv < bv
                bv = jnp.where(lt, dv, bv)
                bc = jnp.where(lt, c, bc)
        gidx = bc * 128 + lane
        lm = jnp.min(bv, axis=1)
        ma = jnp.where(bv == lm[:, None], gidx, N_CHUNK)
        mins.append(lm)
        args.append(jnp.min(ma, axis=1))
    local_min = jnp.concatenate(mins)
    local_arg = jnp.concatenate(args)

    @pl.when(n == 0)
    def _():
        minv_ref[...] = local_min
        mini_ref[...] = local_arg

    @pl.when(n == 1)
    def _():
        m1 = minv_ref[...]
        carry = m1.astype(jnp.bfloat16).astype(jnp.float32)
        take2 = local_min < carry
        idx_ref[...] = jnp.where(take2, local_arg + N_CHUNK, mini_ref[...])
        s = jnp.sum(jnp.where(take2, local_min, m1))

        @pl.when(pl.program_id(0) == 0)
        def _():
            sum_ref[0, 0] = s

        @pl.when(pl.program_id(0) > 0)
        def _():
            sum_ref[0, 0] += s


def _argmin_call(zf, cbt_bf16, csq):
    m_tiles = zf.shape[0] // M_TILE
    return pl.pallas_call(
        _argmin_body,
        grid=(m_tiles, NUM_EMB // N_CHUNK),
        in_specs=[
            pl.BlockSpec((M_TILE, DIM), lambda m, n: (m, 0)),
            pl.BlockSpec((DIM, N_CHUNK), lambda m, n: (0, n)),
            pl.BlockSpec((8, N_CHUNK), lambda m, n: (0, n)),
            pl.BlockSpec((DIM, M_TILE), lambda m, n: (0, m)),
        ],
        out_specs=[
            pl.BlockSpec((M_TILE,), lambda m, n: (m,)),
            pl.BlockSpec(memory_space=pltpu.SMEM),
        ],
        out_shape=[
            jax.ShapeDtypeStruct((zf.shape[0],), jnp.int32),
            jax.ShapeDtypeStruct((1, 1), jnp.float32),
        ],
        scratch_shapes=[
            pltpu.VMEM((M_TILE,), jnp.float32),
            pltpu.VMEM((M_TILE,), jnp.int32),
            pltpu.VMEM((M_TILE, 128), jnp.float32),
        ],
    )(zf, cbt_bf16, csq, zf.T)


def _gather_rows(codebook, idx):
    b = idx.shape[0]
    b_per_w = b // _NW
    mesh = plsc.VectorSubcoreMesh(core_axis_name="c", subcore_axis_name="s")

    @functools.partial(
        pl.kernel,
        out_type=jax.ShapeDtypeStruct((b, DIM), jnp.float32),
        mesh=mesh,
        scratch_types=[
            pltpu.VMEM((b_per_w,), jnp.int32),
            pltpu.VMEM((b_per_w, DIM), jnp.float32),
            pltpu.SemaphoreType.DMA,
        ],
        compiler_params=pltpu.CompilerParams(use_tc_tiling_on_sc=False),
    )
    def gk(cb_hbm, idx_hbm, out_hbm, idx_v, rows_v, sem):
        wid = lax.axis_index("s") * _NC + lax.axis_index("c")
        base = wid * b_per_w
        pltpu.sync_copy(idx_hbm.at[pl.ds(base, b_per_w)], idx_v)
        for c in range(b_per_w // _IDX_CHUNK):
            pltpu.async_copy(
                cb_hbm.at[idx_v.at[pl.ds(c * _IDX_CHUNK, _IDX_CHUNK)]],
                rows_v.at[pl.ds(c * _IDX_CHUNK, _IDX_CHUNK)],
                sem,
            ).wait()
        pltpu.sync_copy(rows_v, out_hbm.at[pl.ds(base, b_per_w)])

    return gk(codebook, idx)


def kernel(z, codebook):
    b, l, d = z.shape
    zf = z.reshape(-1, d)
    csq = jnp.broadcast_to(
        jnp.sum(codebook * codebook, axis=1)[None, :], (8, NUM_EMB))
    idx_flat, dist_sum = _argmin_call(
        zf, codebook.T.astype(jnp.bfloat16), csq)
    z_q = _gather_rows(codebook, idx_flat).reshape(b, l, d)
    vq_loss = 1.25 * (dist_sum[0, 0] / (b * l * d))
    return (z_q, idx_flat.reshape(b, l), vq_loss)
